# Initial kernel scaffold; baseline (speedup 1.0000x reference)
import jax
import jax.numpy as jnp
from jax.experimental import pallas as pl


def kernel(x_paper, x_author, edge_index_cites, edge_index_writes, batch_paper, batch_author, params):
    raise NotImplementedError("write your pallas kernel here")


# speedup vs baseline: 1.1407x; 1.1407x over previous
import functools
import math

import jax
import jax.numpy as jnp
from jax.experimental import pallas as pl
from jax.experimental.pallas import tpu as pltpu

NTS = ['paper', 'author']
ETS = [('paper', 'cites', 'paper'), ('author', 'writes', 'paper')]
H = 8
DH = 32
HID = 256


def _composite(lp):
    out = {}
    for (src, rel, dst) in ETS:
        scale = lp['p_rel'][rel] / math.sqrt(DH)
        a = lp['a_rel'][rel] * scale[:, None, None]
        m = lp['m_rel'][rel]
        fin = lp['W_k'][src].shape[0]
        Wka = jnp.einsum('ihd,hdf->ihf', lp['W_k'][src].reshape(fin, H, DH), a).reshape(fin, HID)
        bka = jnp.einsum('hd,hdf->hf', lp['b_k'][src].reshape(H, DH), a).reshape(HID)
        Wvm = jnp.einsum('ihd,hdf->ihf', lp['W_v'][src].reshape(fin, H, DH), m).reshape(fin, HID)
        bvm = jnp.einsum('hd,hdf->hf', lp['b_v'][src].reshape(H, DH), m).reshape(HID)
        out[rel] = (Wka, bka, Wvm, bvm)
    return out


def _matmul_kernel(x_ref, w_ref, b_ref, o_ref):
    o_ref[...] = jnp.dot(x_ref[...], w_ref[...],
                         preferred_element_type=jnp.float32) + b_ref[...]


def _matmul(x, w, b, bm=1000):
    n, fi = x.shape
    fo = w.shape[1]
    grid = (n // bm,)
    return pl.pallas_call(
        _matmul_kernel,
        grid=grid,
        in_specs=[pl.BlockSpec((bm, fi), lambda i: (i, 0)),
                  pl.BlockSpec((fi, fo), lambda i: (0, 0)),
                  pl.BlockSpec((1, fo), lambda i: (0, 0))],
        out_specs=pl.BlockSpec((bm, fo), lambda i: (i, 0)),
        out_shape=jax.ShapeDtypeStruct((n, fo), jnp.float32),
    )(x, w, b.reshape(1, fo))


def _layer(xd, eid, lp, comp):
    q = {nt: xd[nt] @ lp['W_q'][nt] + lp['b_q'][nt] for nt in NTS}
    agg = {nt: jnp.zeros((xd[nt].shape[0], HID), jnp.float32) for nt in NTS}
    for (src, rel, dst) in ETS:
        Wka, bka, Wvm, bvm = comp[rel]
        ks = xd[src] @ Wka + bka
        vs = xd[src] @ Wvm + bvm
        s_idx, d_idx = eid[rel][0], eid[rel][1]
        kj = ks[s_idx]
        vj = vs[s_idx]
        qi = q[dst][d_idx]
        alpha = (kj * qi).reshape(-1, H, DH).sum(-1)
        e = jnp.exp(alpha)
        n_dst = xd[dst].shape[0]
        den = jax.ops.segment_sum(e, d_idx, num_segments=n_dst)
        msg = vj.reshape(-1, H, DH) * e[:, :, None]
        num = jax.ops.segment_sum(msg, d_idx, num_segments=n_dst)
        agg[dst] = agg[dst] + (num / (den[:, :, None] + 1e-16)).reshape(-1, HID)
    newx = {}
    for nt in NTS:
        o = jax.nn.gelu(agg[nt], approximate=False)
        o = o @ lp['W_o'][nt] + lp['b_o'][nt]
        al = jax.nn.sigmoid(lp['skip'][nt])
        newx[nt] = al * o + (1.0 - al) * xd[nt]
    return newx


def _graph_ln(x, w, b, eps=1e-5):
    mu = jnp.mean(x)
    var = jnp.mean((x - mu) ** 2)
    return (x - mu) / (jnp.sqrt(var) + eps) * w + b


def kernel(x_paper, x_author, edge_index_cites, edge_index_writes,
           batch_paper, batch_author, params):
    xd = {'paper': x_paper, 'author': x_author}
    eid = {'cites': edge_index_cites, 'writes': edge_index_writes}
    xs = {nt: [] for nt in NTS}
    L = len(params['layers'])
    for l in range(L):
        lp = params['layers'][l]
        comp = _composite(lp)
        xd = _layer(xd, eid, lp, comp)
        if l != L - 1:
            xd = {nt: _graph_ln(xd[nt], params['norm_w'][nt], params['norm_b'][nt])
                  for nt in NTS}
        for nt in NTS:
            xs[nt].append(xd[nt])
    outs = []
    for nt in NTS:
        cat = jnp.concatenate(xs[nt], axis=-1)
        outs.append(_matmul(cat, params['out_W'][nt], params['out_b'][nt]))
    return tuple(outs)


# speedup vs baseline: 8.6403x; 7.5745x over previous
import functools
import math

import jax
import jax.numpy as jnp
from jax import lax
from jax.experimental import pallas as pl
from jax.experimental.pallas import tpu as pltpu

NTS = ['paper', 'author']
ETS = [('paper', 'cites', 'paper'), ('author', 'writes', 'paper')]
H = 8
DH = 32
HID = 256
EPS = 1e-16

BM = 1000
BE = 1000


def _proj_p_kernel(x_ref, w_ref, b_ref, sv_ref, q_ref):
    r = jnp.dot(x_ref[...], w_ref[...], preferred_element_type=jnp.float32)
    r = r + b_ref[...]
    sv_ref[...] = r[:, :512]
    q_ref[...] = r[:, 512:]


def _proj_p(x, wcat, bcat):
    n, fi = x.shape
    return pl.pallas_call(
        _proj_p_kernel,
        grid=(n // BM,),
        in_specs=[pl.BlockSpec((BM, fi), lambda i: (i, 0)),
                  pl.BlockSpec((fi, 768), lambda i: (0, 0)),
                  pl.BlockSpec((1, 768), lambda i: (0, 0))],
        out_specs=[pl.BlockSpec((BM, 512), lambda i: (i, 0)),
                   pl.BlockSpec((BM, 256), lambda i: (i, 0))],
        out_shape=[jax.ShapeDtypeStruct((n, 512), jnp.float32),
                   jax.ShapeDtypeStruct((n, 256), jnp.float32)],
    )(x, wcat, bcat.reshape(1, 768))


def _matmul_kernel(x_ref, w_ref, b_ref, o_ref):
    o_ref[...] = jnp.dot(x_ref[...], w_ref[...],
                         preferred_element_type=jnp.float32) + b_ref[...]


def _matmul(x, w, b):
    n, fi = x.shape
    fo = w.shape[1]
    return pl.pallas_call(
        _matmul_kernel,
        grid=(n // BM,),
        in_specs=[pl.BlockSpec((BM, fi), lambda i: (i, 0)),
                  pl.BlockSpec((fi, fo), lambda i: (0, 0)),
                  pl.BlockSpec((1, fo), lambda i: (0, 0))],
        out_specs=pl.BlockSpec((BM, fo), lambda i: (i, 0)),
        out_shape=jax.ShapeDtypeStruct((n, fo), jnp.float32),
    )(x, w, b.reshape(1, fo))


def _msg_kernel(kjv_ref, qi_ref, msg_ref, p_ref):
    kj = kjv_ref[:, :256]
    vj = kjv_ref[:, 256:]
    t = kj * qi_ref[...]
    f = lax.broadcasted_iota(jnp.int32, (256, 16), 0)
    h = lax.broadcasted_iota(jnp.int32, (256, 16), 1)
    b1 = jnp.where((f // DH) == h, 1.0, 0.0)
    alpha = jnp.dot(t, b1, preferred_element_type=jnp.float32)
    e = jnp.exp(alpha)
    h2 = lax.broadcasted_iota(jnp.int32, (16, 256), 0)
    f2 = lax.broadcasted_iota(jnp.int32, (16, 256), 1)
    b2 = jnp.where((f2 // DH) == h2, 1.0, 0.0)
    msg_ref[...] = vj * jnp.dot(e, b2, preferred_element_type=jnp.float32)
    p_ref[...] = e


def _msg(kjv, qi):
    m = kjv.shape[0]
    return pl.pallas_call(
        _msg_kernel,
        grid=(m // BE,),
        in_specs=[pl.BlockSpec((BE, 512), lambda i: (i, 0)),
                  pl.BlockSpec((BE, 256), lambda i: (i, 0))],
        out_specs=[pl.BlockSpec((BE, 256), lambda i: (i, 0)),
                   pl.BlockSpec((BE, 16), lambda i: (i, 0))],
        out_shape=[jax.ShapeDtypeStruct((m, 256), jnp.float32),
                   jax.ShapeDtypeStruct((m, 16), jnp.float32)],
    )(kjv, qi)


def _post_kernel(nc_ref, dc_ref, nw_ref, dw_ref, xp_ref, wo_ref, bo_ref,
                 cp_ref, xa_ref, ca_ref, da_ref, op_ref, oa_ref):
    h2 = lax.broadcasted_iota(jnp.int32, (16, 256), 0)
    f2 = lax.broadcasted_iota(jnp.int32, (16, 256), 1)
    rep = jnp.where((f2 // DH) == h2, 1.0, 0.0)
    den_c = jnp.dot(dc_ref[...], rep, preferred_element_type=jnp.float32)
    den_w = jnp.dot(dw_ref[...], rep, preferred_element_type=jnp.float32)
    agg = nc_ref[...] / (den_c + EPS) + nw_ref[...] / (den_w + EPS)
    g = 0.5 * agg * (1.0 + lax.erf(agg / math.sqrt(2.0)))
    op_ref[...] = (jnp.dot(g, wo_ref[...], preferred_element_type=jnp.float32)
                   + bo_ref[...] + xp_ref[...] * cp_ref[...])
    oa_ref[...] = xa_ref[...] * ca_ref[...] + da_ref[...]


def _post(num_c, den_c, num_w, den_w, xp, wo, bo, cp, xa, ca, da):
    n = xp.shape[0]
    row = pl.BlockSpec((BM, 256), lambda i: (i, 0))
    small = pl.BlockSpec((BM, 16), lambda i: (i, 0))
    const = pl.BlockSpec((1, 256), lambda i: (0, 0))
    return pl.pallas_call(
        _post_kernel,
        grid=(n // BM,),
        in_specs=[row, small, row, small, row,
                  pl.BlockSpec((256, 256), lambda i: (0, 0)), const,
                  const, row, const, const],
        out_specs=[row, row],
        out_shape=[jax.ShapeDtypeStruct((n, 256), jnp.float32),
                   jax.ShapeDtypeStruct((n, 256), jnp.float32)],
    )(num_c, den_c, num_w, den_w, xp, wo, bo.reshape(1, 256),
      cp, xa, ca, da)


def _ln_stats_kernel(x_ref, o_ref):
    @pl.when(pl.program_id(0) == 0)
    def _():
        o_ref[...] = jnp.zeros((1, 128), jnp.float32)
    s = jnp.sum(x_ref[...])
    ss = jnp.sum(x_ref[...] * x_ref[...])
    lanes = lax.broadcasted_iota(jnp.int32, (1, 128), 1)
    row = jnp.where(lanes == 0, s, 0.0) + jnp.where(lanes == 1, ss, 0.0)
    o_ref[...] += row


def _ln_apply_kernel(x_ref, st_ref, w_ref, b_ref, o_ref, *, count):
    s = st_ref[0, 0]
    ss = st_ref[0, 1]
    mu = s / count
    var = ss / count - mu * mu
    inv = 1.0 / (jnp.sqrt(var) + 1e-5)
    o_ref[...] = (x_ref[...] - mu) * inv * w_ref[...] + b_ref[...]


def _graph_ln(x, w, b):
    n = x.shape[0]
    g = n // BM
    stats = pl.pallas_call(
        _ln_stats_kernel,
        grid=(g,),
        in_specs=[pl.BlockSpec((BM, 256), lambda i: (i, 0))],
        out_specs=pl.BlockSpec((1, 128), lambda i: (0, 0)),
        out_shape=jax.ShapeDtypeStruct((1, 128), jnp.float32),
    )(x)
    return pl.pallas_call(
        functools.partial(_ln_apply_kernel, count=float(n * 256)),
        grid=(g,),
        in_specs=[pl.BlockSpec((BM, 256), lambda i: (i, 0)),
                  pl.BlockSpec((1, 128), lambda i: (0, 0)),
                  pl.BlockSpec((1, 256), lambda i: (0, 0)),
                  pl.BlockSpec((1, 256), lambda i: (0, 0))],
        out_specs=pl.BlockSpec((BM, 256), lambda i: (i, 0)),
        out_shape=jax.ShapeDtypeStruct((n, 256), jnp.float32),
    )(x, stats, w.reshape(1, 256), b.reshape(1, 256))


def _heads_kernel(x0_ref, x1_ref, w0_ref, w1_ref, b_ref, o_ref):
    o_ref[...] = (jnp.dot(x0_ref[...], w0_ref[...], preferred_element_type=jnp.float32)
                  + jnp.dot(x1_ref[...], w1_ref[...], preferred_element_type=jnp.float32)
                  + b_ref[...])


def _heads(x0, x1, w, b):
    n = x0.shape[0]
    w0, w1 = w[:256], w[256:]
    fo = w.shape[1]
    return pl.pallas_call(
        _heads_kernel,
        grid=(n // BM,),
        in_specs=[pl.BlockSpec((BM, 256), lambda i: (i, 0)),
                  pl.BlockSpec((BM, 256), lambda i: (i, 0)),
                  pl.BlockSpec((256, fo), lambda i: (0, 0)),
                  pl.BlockSpec((256, fo), lambda i: (0, 0)),
                  pl.BlockSpec((1, fo), lambda i: (0, 0))],
        out_specs=pl.BlockSpec((BM, fo), lambda i: (i, 0)),
        out_shape=jax.ShapeDtypeStruct((n, fo), jnp.float32),
    )(x0, x1, w0, w1, b.reshape(1, fo))


def _edge_gather(svp, svw, qp, s_c, d_c, s_w, d_w):
    kjv = jnp.concatenate([svp[s_c], svw[s_w]], axis=0)
    qi = jnp.concatenate([qp[d_c], qp[d_w]], axis=0)
    return kjv, qi


def _edge_scatter(msg, p, d_idx, n):
    num = jax.ops.segment_sum(msg, d_idx, num_segments=n)
    den = jax.ops.segment_sum(p, d_idx, num_segments=n)
    return num, den


def _assemble(params):
    pre = {'layers': []}
    for lp in params['layers']:
        comp = {}
        for (src, rel, dst) in ETS:
            scale = lp['p_rel'][rel] / math.sqrt(DH)
            a = lp['a_rel'][rel] * scale[:, None, None]
            m = lp['m_rel'][rel]
            fin = lp['W_k'][src].shape[0]
            wka = jnp.einsum('ihd,hdf->ihf', lp['W_k'][src].reshape(fin, H, DH), a).reshape(fin, HID)
            bka = jnp.einsum('hd,hdf->hf', lp['b_k'][src].reshape(H, DH), a).reshape(HID)
            wvm = jnp.einsum('ihd,hdf->ihf', lp['W_v'][src].reshape(fin, H, DH), m).reshape(fin, HID)
            bvm = jnp.einsum('hd,hdf->hf', lp['b_v'][src].reshape(H, DH), m).reshape(HID)
            comp[rel] = (wka, bka, wvm, bvm)
        wcat_p = jnp.concatenate([comp['cites'][0], comp['cites'][2],
                                  lp['W_q']['paper']], axis=1)
        bcat_p = jnp.concatenate([comp['cites'][1], comp['cites'][3],
                                  lp['b_q']['paper']])
        wcat_a = jnp.concatenate([comp['writes'][0], comp['writes'][2]], axis=1)
        bcat_a = jnp.concatenate([comp['writes'][1], comp['writes'][3]])
        sig_p = jax.nn.sigmoid(lp['skip']['paper'])
        sig_a = jax.nn.sigmoid(lp['skip']['author'])
        ones = jnp.ones((1, 256), jnp.float32)
        pre['layers'].append({
            'wcat_p': wcat_p, 'bcat_p': bcat_p,
            'wcat_a': wcat_a, 'bcat_a': bcat_a,
            'wo_p': lp['W_o']['paper'] * sig_p,
            'bo_p': lp['b_o']['paper'] * sig_p,
            'cp': ones * (1.0 - sig_p),
            'ca': ones * (1.0 - sig_a),
            'da': (lp['b_o']['author'] * sig_a).reshape(1, 256),
        })
    return pre


def kernel(x_paper, x_author, edge_index_cites, edge_index_writes,
           batch_paper, batch_author, params):
    n = x_paper.shape[0]
    e = edge_index_cites.shape[1]
    s_c, d_c = edge_index_cites[0], edge_index_cites[1]
    s_w, d_w = edge_index_writes[0], edge_index_writes[1]
    pre = _assemble(params)

    xp, xa = x_paper, x_author
    xs_p, xs_a = [], []
    nl = len(params['layers'])
    for l in range(nl):
        w = pre['layers'][l]
        svp, qp = _proj_p(xp, w['wcat_p'], w['bcat_p'])
        svw = _matmul(xa, w['wcat_a'], w['bcat_a'])
        kjv, qi = _edge_gather(svp, svw, qp, s_c, d_c, s_w, d_w)
        msg, p = _msg(kjv, qi)
        num_c, den_c = _edge_scatter(msg[:e], p[:e], d_c, n)
        num_w, den_w = _edge_scatter(msg[e:], p[e:], d_w, n)
        xp, xa = _post(num_c, den_c, num_w, den_w, xp,
                       w['wo_p'], w['bo_p'], w['cp'], xa, w['ca'], w['da'])
        if l != nl - 1:
            xp = _graph_ln(xp, params['norm_w']['paper'], params['norm_b']['paper'])
            xa = _graph_ln(xa, params['norm_w']['author'], params['norm_b']['author'])
        xs_p.append(xp)
        xs_a.append(xa)

    out_p = _heads(xs_p[0], xs_p[1], params['out_W']['paper'], params['out_b']['paper'])
    out_a = _heads(xs_a[0], xs_a[1], params['out_W']['author'], params['out_b']['author'])
    return (out_p, out_a)


# speedup vs baseline: 12.5418x; 1.4516x over previous
import functools
import math

import jax
import jax.numpy as jnp
from jax import lax
from jax.experimental import pallas as pl
from jax.experimental.pallas import tpu as pltpu

NTS = ['paper', 'author']
ETS = [('paper', 'cites', 'paper'), ('author', 'writes', 'paper')]
H = 8
DH = 32
HID = 256
EPS = 1e-16

BM = 1000
BE = 1000


def _proj_p_kernel(x_ref, w_ref, b_ref, sv_ref, q_ref):
    r = jnp.dot(x_ref[...], w_ref[...], preferred_element_type=jnp.float32)
    r = r + b_ref[...]
    sv_ref[...] = r[:, :512]
    q_ref[...] = r[:, 512:]


def _proj_p(x, wcat, bcat):
    n, fi = x.shape
    return pl.pallas_call(
        _proj_p_kernel,
        grid=(n // BM,),
        in_specs=[pl.BlockSpec((BM, fi), lambda i: (i, 0)),
                  pl.BlockSpec((fi, 768), lambda i: (0, 0)),
                  pl.BlockSpec((1, 768), lambda i: (0, 0))],
        out_specs=[pl.BlockSpec((BM, 512), lambda i: (i, 0)),
                   pl.BlockSpec((BM, 256), lambda i: (i, 0))],
        out_shape=[jax.ShapeDtypeStruct((n, 512), jnp.float32),
                   jax.ShapeDtypeStruct((n, 256), jnp.float32)],
    )(x, wcat, bcat.reshape(1, 768))


def _matmul_kernel(x_ref, w_ref, b_ref, o_ref):
    o_ref[...] = jnp.dot(x_ref[...], w_ref[...],
                         preferred_element_type=jnp.float32) + b_ref[...]


def _matmul(x, w, b):
    n, fi = x.shape
    fo = w.shape[1]
    return pl.pallas_call(
        _matmul_kernel,
        grid=(n // BM,),
        in_specs=[pl.BlockSpec((BM, fi), lambda i: (i, 0)),
                  pl.BlockSpec((fi, fo), lambda i: (0, 0)),
                  pl.BlockSpec((1, fo), lambda i: (0, 0))],
        out_specs=pl.BlockSpec((BM, fo), lambda i: (i, 0)),
        out_shape=jax.ShapeDtypeStruct((n, fo), jnp.float32),
    )(x, w, b.reshape(1, fo))


def _msg_kernel(kjv_ref, qi_ref, mp_ref):
    kj = kjv_ref[:, :256]
    vj = kjv_ref[:, 256:]
    t = kj * qi_ref[...]
    f = lax.broadcasted_iota(jnp.int32, (256, 16), 0)
    h = lax.broadcasted_iota(jnp.int32, (256, 16), 1)
    b1 = jnp.where((f // DH) == h, 1.0, 0.0)
    alpha = jnp.dot(t, b1, preferred_element_type=jnp.float32)
    e = jnp.exp(alpha)
    h2 = lax.broadcasted_iota(jnp.int32, (16, 256), 0)
    f2 = lax.broadcasted_iota(jnp.int32, (16, 256), 1)
    b2 = jnp.where((f2 // DH) == h2, 1.0, 0.0)
    h3 = lax.broadcasted_iota(jnp.int32, (16, 128), 0)
    f3 = lax.broadcasted_iota(jnp.int32, (16, 128), 1)
    b3 = jnp.where(f3 == h3, 1.0, 0.0)
    msg = vj * jnp.dot(e, b2, preferred_element_type=jnp.float32)
    e128 = jnp.dot(e, b3, preferred_element_type=jnp.float32)
    mp_ref[...] = jnp.concatenate([msg, e128], axis=1)


def _msg(kjv, qi):
    m = kjv.shape[0]
    return pl.pallas_call(
        _msg_kernel,
        grid=(m // BE,),
        in_specs=[pl.BlockSpec((BE, 512), lambda i: (i, 0)),
                  pl.BlockSpec((BE, 256), lambda i: (i, 0))],
        out_specs=pl.BlockSpec((BE, 384), lambda i: (i, 0)),
        out_shape=jax.ShapeDtypeStruct((m, 384), jnp.float32),
    )(kjv, qi)


def _post_kernel(nc_ref, nw_ref, xp_ref, wo_ref, bo_ref,
                 cp_ref, xa_ref, ca_ref, da_ref, op_ref, oa_ref):
    h2 = lax.broadcasted_iota(jnp.int32, (128, 256), 0)
    f2 = lax.broadcasted_iota(jnp.int32, (128, 256), 1)
    rep = jnp.where((f2 // DH) == h2, 1.0, 0.0)
    den_c = jnp.dot(nc_ref[:, 256:], rep, preferred_element_type=jnp.float32)
    den_w = jnp.dot(nw_ref[:, 256:], rep, preferred_element_type=jnp.float32)
    agg = nc_ref[:, :256] / (den_c + EPS) + nw_ref[:, :256] / (den_w + EPS)
    g = 0.5 * agg * (1.0 + lax.erf(agg / math.sqrt(2.0)))
    op_ref[...] = (jnp.dot(g, wo_ref[...], preferred_element_type=jnp.float32)
                   + bo_ref[...] + xp_ref[...] * cp_ref[...])
    oa_ref[...] = xa_ref[...] * ca_ref[...] + da_ref[...]


def _post(nd_c, nd_w, xp, wo, bo, cp, xa, ca, da):
    n = xp.shape[0]
    row = pl.BlockSpec((BM, 256), lambda i: (i, 0))
    wide = pl.BlockSpec((BM, 384), lambda i: (i, 0))
    const = pl.BlockSpec((1, 256), lambda i: (0, 0))
    return pl.pallas_call(
        _post_kernel,
        grid=(n // BM,),
        in_specs=[wide, wide, row,
                  pl.BlockSpec((256, 256), lambda i: (0, 0)), const,
                  const, row, const, const],
        out_specs=[row, row],
        out_shape=[jax.ShapeDtypeStruct((n, 256), jnp.float32),
                   jax.ShapeDtypeStruct((n, 256), jnp.float32)],
    )(nd_c, nd_w, xp, wo, bo.reshape(1, 256),
      cp, xa, ca, da)


def _ln_stats_kernel(x_ref, o_ref):
    @pl.when(pl.program_id(0) == 0)
    def _():
        o_ref[...] = jnp.zeros((1, 128), jnp.float32)
    s = jnp.sum(x_ref[...])
    ss = jnp.sum(x_ref[...] * x_ref[...])
    lanes = lax.broadcasted_iota(jnp.int32, (1, 128), 1)
    row = jnp.where(lanes == 0, s, 0.0) + jnp.where(lanes == 1, ss, 0.0)
    o_ref[...] += row


def _ln_apply_kernel(x_ref, st_ref, w_ref, b_ref, o_ref, *, count):
    s = st_ref[0, 0]
    ss = st_ref[0, 1]
    mu = s / count
    var = ss / count - mu * mu
    inv = 1.0 / (jnp.sqrt(var) + 1e-5)
    o_ref[...] = (x_ref[...] - mu) * inv * w_ref[...] + b_ref[...]


def _graph_ln(x, w, b):
    n = x.shape[0]
    g = n // BM
    stats = pl.pallas_call(
        _ln_stats_kernel,
        grid=(g,),
        in_specs=[pl.BlockSpec((BM, 256), lambda i: (i, 0))],
        out_specs=pl.BlockSpec((1, 128), lambda i: (0, 0)),
        out_shape=jax.ShapeDtypeStruct((1, 128), jnp.float32),
    )(x)
    return pl.pallas_call(
        functools.partial(_ln_apply_kernel, count=float(n * 256)),
        grid=(g,),
        in_specs=[pl.BlockSpec((BM, 256), lambda i: (i, 0)),
                  pl.BlockSpec((1, 128), lambda i: (0, 0)),
                  pl.BlockSpec((1, 256), lambda i: (0, 0)),
                  pl.BlockSpec((1, 256), lambda i: (0, 0))],
        out_specs=pl.BlockSpec((BM, 256), lambda i: (i, 0)),
        out_shape=jax.ShapeDtypeStruct((n, 256), jnp.float32),
    )(x, stats, w.reshape(1, 256), b.reshape(1, 256))


def _heads_kernel(x0_ref, x1_ref, w0_ref, w1_ref, b_ref, o_ref):
    o_ref[...] = (jnp.dot(x0_ref[...], w0_ref[...], preferred_element_type=jnp.float32)
                  + jnp.dot(x1_ref[...], w1_ref[...], preferred_element_type=jnp.float32)
                  + b_ref[...])


def _heads(x0, x1, w, b):
    n = x0.shape[0]
    w0, w1 = w[:256], w[256:]
    fo = w.shape[1]
    return pl.pallas_call(
        _heads_kernel,
        grid=(n // BM,),
        in_specs=[pl.BlockSpec((BM, 256), lambda i: (i, 0)),
                  pl.BlockSpec((BM, 256), lambda i: (i, 0)),
                  pl.BlockSpec((256, fo), lambda i: (0, 0)),
                  pl.BlockSpec((256, fo), lambda i: (0, 0)),
                  pl.BlockSpec((1, fo), lambda i: (0, 0))],
        out_specs=pl.BlockSpec((BM, fo), lambda i: (i, 0)),
        out_shape=jax.ShapeDtypeStruct((n, fo), jnp.float32),
    )(x0, x1, w0, w1, b.reshape(1, fo))


GCH = 40
SV_GRP = 5
QI_GRP = 2
SCH = 80
SLAB = 5120
DUMMY = 5100


def _gather_job(idx_h, table_h, out_h, arr_base, out_base, idxbuf, rowbuf,
                sem, n_rows, grp):
    rows_per_group = grp * GCH
    ngroups = n_rows // rows_per_group

    def body(g, _):
        goff = g * rows_per_group
        pltpu.sync_copy(idx_h.at[pl.ds(arr_base + goff, rows_per_group)], idxbuf)
        descs = [pltpu.async_copy(table_h.at[idxbuf.at[pl.ds(b * GCH, GCH)]],
                                  rowbuf.at[pl.ds(b * GCH, GCH)], sem)
                 for b in range(grp)]
        for dsc in descs:
            dsc.wait()
        pltpu.sync_copy(rowbuf, out_h.at[pl.ds(out_base + goff, rows_per_group)])
        return 0

    lax.fori_loop(0, ngroups, body, 0)


def _sc_gather(svp, svw, qp, s_c, d_c, s_w, d_w):
    import jax.experimental.pallas.tpu_sc as plsc
    e = s_c.shape[0]
    info = plsc.get_sparse_core_info()
    ns = info.num_subcores
    epw = e // ns

    @functools.partial(
        pl.kernel,
        out_type=[jax.ShapeDtypeStruct((2 * e, 512), jnp.float32),
                  jax.ShapeDtypeStruct((2 * e, 256), jnp.float32)],
        mesh=plsc.VectorSubcoreMesh(core_axis_name="c", subcore_axis_name="s"),
        scratch_types=[
            pltpu.VMEM((SV_GRP * GCH,), jnp.int32),
            pltpu.VMEM((SV_GRP * GCH, 512), jnp.float32),
            pltpu.VMEM((QI_GRP * GCH,), jnp.int32),
            pltpu.VMEM((QI_GRP * GCH, 256), jnp.float32),
            pltpu.SemaphoreType.DMA,
        ],
    )
    def k(svp_h, svw_h, qp_h, sc_h, dc_h, sw_h, dw_h, kjv_h, qi_h,
          idx1, rows1, idx2, rows2, sem):
        c = lax.axis_index("c")
        s = lax.axis_index("s")
        base = s * epw

        @pl.when(c == 0)
        def _():
            _gather_job(sc_h, svp_h, kjv_h, base, base, idx1, rows1, sem,
                        epw, SV_GRP)
            _gather_job(dc_h, qp_h, qi_h, base, base, idx2, rows2, sem,
                        epw, QI_GRP)

        @pl.when(c == 1)
        def _():
            _gather_job(sw_h, svw_h, kjv_h, base, e + base, idx1, rows1, sem,
                        epw, SV_GRP)
            _gather_job(dw_h, qp_h, qi_h, base, e + base, idx2, rows2, sem,
                        epw, QI_GRP)

    return k(svp, svw, qp, s_c, d_c, s_w, d_w)


def _sc_scatter(msgp, d_c, d_w, n):
    import jax.experimental.pallas.tpu_sc as plsc
    e = d_c.shape[0]
    info = plsc.get_sparse_core_info()
    ns = info.num_subcores
    ept = e // ns
    zb = 1000

    @functools.partial(
        pl.kernel,
        out_type=[jax.ShapeDtypeStruct((n, 384), jnp.float32),
                  jax.ShapeDtypeStruct((n, 384), jnp.float32)],
        mesh=plsc.VectorSubcoreMesh(core_axis_name="c", subcore_axis_name="s"),
        scratch_types=[
            pltpu.VMEM((SCH,), jnp.int32),
            pltpu.VMEM((SCH, 384), jnp.float32),
            [pltpu.VMEM((16,), jnp.int32) for _ in range(SCH // 16)],
        ],
    )
    def k(mp_h, dc_h, dw_h, z_h, ndc_h, ndw_h, idxbuf, rowbuf, idxs):
        c = lax.axis_index("c")
        s = lax.axis_index("s")

        def job(d_h, out_h, eoff):
            @pl.when(s < n // zb)
            def _():
                pltpu.sync_copy(z_h, out_h.at[pl.ds(s * zb, zb)])

            plsc.subcore_barrier()
            base = s * ept

            def body(i, _):
                off = base + i * SCH
                pltpu.sync_copy(d_h.at[pl.ds(off, SCH)], idxbuf)
                pltpu.sync_copy(mp_h.at[pl.ds(eoff + off, SCH)], rowbuf)
                for j in range(SCH // 16):
                    idxs[j][...] = idxbuf[pl.ds(j * 16, 16)]
                    pltpu.sync_copy(rowbuf.at[pl.ds(j * 16, 16)],
                                    out_h.at[idxs[j]], add=True)
                return 0

            @pl.when(s == 0)
            def _():
                lax.fori_loop(0, ept // SCH, body, 0)

        @pl.when(c == 0)
        def _():
            job(dc_h, ndc_h, 0)

        @pl.when(c == 1)
        def _():
            job(dw_h, ndw_h, e)

    z = jnp.zeros((zb, 384), jnp.float32)
    return k(msgp, d_c, d_w, z)


def _assemble(params):
    pre = {'layers': []}
    for lp in params['layers']:
        comp = {}
        for (src, rel, dst) in ETS:
            scale = lp['p_rel'][rel] / math.sqrt(DH)
            a = lp['a_rel'][rel] * scale[:, None, None]
            m = lp['m_rel'][rel]
            fin = lp['W_k'][src].shape[0]
            wka = jnp.einsum('ihd,hdf->ihf', lp['W_k'][src].reshape(fin, H, DH), a).reshape(fin, HID)
            bka = jnp.einsum('hd,hdf->hf', lp['b_k'][src].reshape(H, DH), a).reshape(HID)
            wvm = jnp.einsum('ihd,hdf->ihf', lp['W_v'][src].reshape(fin, H, DH), m).reshape(fin, HID)
            bvm = jnp.einsum('hd,hdf->hf', lp['b_v'][src].reshape(H, DH), m).reshape(HID)
            comp[rel] = (wka, bka, wvm, bvm)
        wcat_p = jnp.concatenate([comp['cites'][0], comp['cites'][2],
                                  lp['W_q']['paper']], axis=1)
        bcat_p = jnp.concatenate([comp['cites'][1], comp['cites'][3],
                                  lp['b_q']['paper']])
        wcat_a = jnp.concatenate([comp['writes'][0], comp['writes'][2]], axis=1)
        bcat_a = jnp.concatenate([comp['writes'][1], comp['writes'][3]])
        sig_p = jax.nn.sigmoid(lp['skip']['paper'])
        sig_a = jax.nn.sigmoid(lp['skip']['author'])
        ones = jnp.ones((1, 256), jnp.float32)
        pre['layers'].append({
            'wcat_p': wcat_p, 'bcat_p': bcat_p,
            'wcat_a': wcat_a, 'bcat_a': bcat_a,
            'wo_p': lp['W_o']['paper'] * sig_p,
            'bo_p': lp['b_o']['paper'] * sig_p,
            'cp': ones * (1.0 - sig_p),
            'ca': ones * (1.0 - sig_a),
            'da': (lp['b_o']['author'] * sig_a).reshape(1, 256),
        })
    return pre


def kernel(x_paper, x_author, edge_index_cites, edge_index_writes,
           batch_paper, batch_author, params):
    n = x_paper.shape[0]
    e = edge_index_cites.shape[1]
    s_c, d_c = edge_index_cites[0], edge_index_cites[1]
    s_w, d_w = edge_index_writes[0], edge_index_writes[1]
    pre = _assemble(params)

    xp, xa = x_paper, x_author
    xs_p, xs_a = [], []
    nl = len(params['layers'])
    for l in range(nl):
        w = pre['layers'][l]
        svp, qp = _proj_p(xp, w['wcat_p'], w['bcat_p'])
        svw = _matmul(xa, w['wcat_a'], w['bcat_a'])
        kjv, qi = _sc_gather(svp, svw, qp, s_c, d_c, s_w, d_w)
        msgp = _msg(kjv, qi)
        nd_c = jax.ops.segment_sum(msgp[:e], d_c, num_segments=n)
        nd_w = jax.ops.segment_sum(msgp[e:], d_w, num_segments=n)
        xp, xa = _post(nd_c, nd_w, xp,
                       w['wo_p'], w['bo_p'], w['cp'], xa, w['ca'], w['da'])
        if l != nl - 1:
            xp = _graph_ln(xp, params['norm_w']['paper'], params['norm_b']['paper'])
            xa = _graph_ln(xa, params['norm_w']['author'], params['norm_b']['author'])
        xs_p.append(xp)
        xs_a.append(xa)

    out_p = _heads(xs_p[0], xs_p[1], params['out_W']['paper'], params['out_b']['paper'])
    out_a = _heads(xs_a[0], xs_a[1], params['out_W']['author'], params['out_b']['author'])
    return (out_p, out_a)


# speedup vs baseline: 12.5450x; 1.0003x over previous
import functools
import math

import jax
import jax.numpy as jnp
from jax import lax
from jax.experimental import pallas as pl
from jax.experimental.pallas import tpu as pltpu

NTS = ['paper', 'author']
ETS = [('paper', 'cites', 'paper'), ('author', 'writes', 'paper')]
H = 8
DH = 32
HID = 256
EPS = 1e-16

BM = 1000
BE = 1000


def _proj_p_kernel(x_ref, w_ref, b_ref, sv_ref, q_ref):
    r = jnp.dot(x_ref[...], w_ref[...], preferred_element_type=jnp.float32)
    r = r + b_ref[...]
    sv_ref[...] = r[:, :512]
    q_ref[...] = r[:, 512:]


def _proj_p(x, wcat, bcat):
    n, fi = x.shape
    return pl.pallas_call(
        _proj_p_kernel,
        grid=(n // BM,),
        in_specs=[pl.BlockSpec((BM, fi), lambda i: (i, 0)),
                  pl.BlockSpec((fi, 768), lambda i: (0, 0)),
                  pl.BlockSpec((1, 768), lambda i: (0, 0))],
        out_specs=[pl.BlockSpec((BM, 512), lambda i: (i, 0)),
                   pl.BlockSpec((BM, 256), lambda i: (i, 0))],
        out_shape=[jax.ShapeDtypeStruct((n, 512), jnp.float32),
                   jax.ShapeDtypeStruct((n, 256), jnp.float32)],
    )(x, wcat, bcat.reshape(1, 768))


def _matmul_kernel(x_ref, w_ref, b_ref, o_ref):
    o_ref[...] = jnp.dot(x_ref[...], w_ref[...],
                         preferred_element_type=jnp.float32) + b_ref[...]


def _matmul(x, w, b):
    n, fi = x.shape
    fo = w.shape[1]
    return pl.pallas_call(
        _matmul_kernel,
        grid=(n // BM,),
        in_specs=[pl.BlockSpec((BM, fi), lambda i: (i, 0)),
                  pl.BlockSpec((fi, fo), lambda i: (0, 0)),
                  pl.BlockSpec((1, fo), lambda i: (0, 0))],
        out_specs=pl.BlockSpec((BM, fo), lambda i: (i, 0)),
        out_shape=jax.ShapeDtypeStruct((n, fo), jnp.float32),
    )(x, w, b.reshape(1, fo))


def _msg_kernel(kjv_ref, qi_ref, mp_ref):
    kj = kjv_ref[:, :256]
    vj = kjv_ref[:, 256:]
    t = kj * qi_ref[...]
    f = lax.broadcasted_iota(jnp.int32, (256, 16), 0)
    h = lax.broadcasted_iota(jnp.int32, (256, 16), 1)
    b1 = jnp.where((f // DH) == h, 1.0, 0.0)
    alpha = jnp.dot(t, b1, preferred_element_type=jnp.float32)
    e = jnp.exp(alpha)
    h2 = lax.broadcasted_iota(jnp.int32, (16, 256), 0)
    f2 = lax.broadcasted_iota(jnp.int32, (16, 256), 1)
    b2 = jnp.where((f2 // DH) == h2, 1.0, 0.0)
    h3 = lax.broadcasted_iota(jnp.int32, (16, 128), 0)
    f3 = lax.broadcasted_iota(jnp.int32, (16, 128), 1)
    b3 = jnp.where(f3 == h3, 1.0, 0.0)
    msg = vj * jnp.dot(e, b2, preferred_element_type=jnp.float32)
    e128 = jnp.dot(e, b3, preferred_element_type=jnp.float32)
    mp_ref[...] = jnp.concatenate([msg, e128], axis=1)


def _msg(kjv, qi):
    m = kjv.shape[0]
    return pl.pallas_call(
        _msg_kernel,
        grid=(m // BE,),
        in_specs=[pl.BlockSpec((BE, 512), lambda i: (i, 0)),
                  pl.BlockSpec((BE, 256), lambda i: (i, 0))],
        out_specs=pl.BlockSpec((BE, 384), lambda i: (i, 0)),
        out_shape=jax.ShapeDtypeStruct((m, 384), jnp.float32),
    )(kjv, qi)


def _post_kernel(nc_ref, nw_ref, xp_ref, wo_ref, bo_ref,
                 cp_ref, xa_ref, ca_ref, da_ref, op_ref, oa_ref):
    h2 = lax.broadcasted_iota(jnp.int32, (128, 256), 0)
    f2 = lax.broadcasted_iota(jnp.int32, (128, 256), 1)
    rep = jnp.where((f2 // DH) == h2, 1.0, 0.0)
    den_c = jnp.dot(nc_ref[:, 256:], rep, preferred_element_type=jnp.float32)
    den_w = jnp.dot(nw_ref[:, 256:], rep, preferred_element_type=jnp.float32)
    agg = nc_ref[:, :256] / (den_c + EPS) + nw_ref[:, :256] / (den_w + EPS)
    g = 0.5 * agg * (1.0 + lax.erf(agg / math.sqrt(2.0)))
    op_ref[...] = (jnp.dot(g, wo_ref[...], preferred_element_type=jnp.float32)
                   + bo_ref[...] + xp_ref[...] * cp_ref[...])
    oa_ref[...] = xa_ref[...] * ca_ref[...] + da_ref[...]


def _post(nd_c, nd_w, xp, wo, bo, cp, xa, ca, da):
    n = xp.shape[0]
    row = pl.BlockSpec((BM, 256), lambda i: (i, 0))
    wide = pl.BlockSpec((BM, 384), lambda i: (i, 0))
    const = pl.BlockSpec((1, 256), lambda i: (0, 0))
    return pl.pallas_call(
        _post_kernel,
        grid=(n // BM,),
        in_specs=[wide, wide, row,
                  pl.BlockSpec((256, 256), lambda i: (0, 0)), const,
                  const, row, const, const],
        out_specs=[row, row],
        out_shape=[jax.ShapeDtypeStruct((n, 256), jnp.float32),
                   jax.ShapeDtypeStruct((n, 256), jnp.float32)],
    )(nd_c, nd_w, xp, wo, bo.reshape(1, 256),
      cp, xa, ca, da)


def _ln_stats_kernel(x_ref, o_ref):
    @pl.when(pl.program_id(0) == 0)
    def _():
        o_ref[...] = jnp.zeros((1, 128), jnp.float32)
    s = jnp.sum(x_ref[...])
    ss = jnp.sum(x_ref[...] * x_ref[...])
    lanes = lax.broadcasted_iota(jnp.int32, (1, 128), 1)
    row = jnp.where(lanes == 0, s, 0.0) + jnp.where(lanes == 1, ss, 0.0)
    o_ref[...] += row


def _ln_apply_kernel(x_ref, st_ref, w_ref, b_ref, o_ref, *, count):
    s = st_ref[0, 0]
    ss = st_ref[0, 1]
    mu = s / count
    var = ss / count - mu * mu
    inv = 1.0 / (jnp.sqrt(var) + 1e-5)
    o_ref[...] = (x_ref[...] - mu) * inv * w_ref[...] + b_ref[...]


def _graph_ln(x, w, b):
    n = x.shape[0]
    g = n // BM
    stats = pl.pallas_call(
        _ln_stats_kernel,
        grid=(g,),
        in_specs=[pl.BlockSpec((BM, 256), lambda i: (i, 0))],
        out_specs=pl.BlockSpec((1, 128), lambda i: (0, 0)),
        out_shape=jax.ShapeDtypeStruct((1, 128), jnp.float32),
    )(x)
    return pl.pallas_call(
        functools.partial(_ln_apply_kernel, count=float(n * 256)),
        grid=(g,),
        in_specs=[pl.BlockSpec((BM, 256), lambda i: (i, 0)),
                  pl.BlockSpec((1, 128), lambda i: (0, 0)),
                  pl.BlockSpec((1, 256), lambda i: (0, 0)),
                  pl.BlockSpec((1, 256), lambda i: (0, 0))],
        out_specs=pl.BlockSpec((BM, 256), lambda i: (i, 0)),
        out_shape=jax.ShapeDtypeStruct((n, 256), jnp.float32),
    )(x, stats, w.reshape(1, 256), b.reshape(1, 256))


def _heads_kernel(x0_ref, x1_ref, w0_ref, w1_ref, b_ref, o_ref):
    o_ref[...] = (jnp.dot(x0_ref[...], w0_ref[...], preferred_element_type=jnp.float32)
                  + jnp.dot(x1_ref[...], w1_ref[...], preferred_element_type=jnp.float32)
                  + b_ref[...])


def _heads(x0, x1, w, b):
    n = x0.shape[0]
    w0, w1 = w[:256], w[256:]
    fo = w.shape[1]
    return pl.pallas_call(
        _heads_kernel,
        grid=(n // BM,),
        in_specs=[pl.BlockSpec((BM, 256), lambda i: (i, 0)),
                  pl.BlockSpec((BM, 256), lambda i: (i, 0)),
                  pl.BlockSpec((256, fo), lambda i: (0, 0)),
                  pl.BlockSpec((256, fo), lambda i: (0, 0)),
                  pl.BlockSpec((1, fo), lambda i: (0, 0))],
        out_specs=pl.BlockSpec((BM, fo), lambda i: (i, 0)),
        out_shape=jax.ShapeDtypeStruct((n, fo), jnp.float32),
    )(x0, x1, w0, w1, b.reshape(1, fo))


GCH = 40
SV_GRP = 5
QI_GRP = 2


def _gather_job(idx_h, table_h, out_h, arr_base, out_base, idxbuf, rowbuf,
                sem, n_rows, grp):
    rows_per_group = grp * GCH
    ngroups = n_rows // rows_per_group

    def body(g, _):
        goff = g * rows_per_group
        pltpu.sync_copy(idx_h.at[pl.ds(arr_base + goff, rows_per_group)], idxbuf)
        descs = [pltpu.async_copy(table_h.at[idxbuf.at[pl.ds(b * GCH, GCH)]],
                                  rowbuf.at[pl.ds(b * GCH, GCH)], sem)
                 for b in range(grp)]
        for dsc in descs:
            dsc.wait()
        pltpu.sync_copy(rowbuf, out_h.at[pl.ds(out_base + goff, rows_per_group)])
        return 0

    lax.fori_loop(0, ngroups, body, 0)


def _sc_gather(svp, svw, qp, s_c, d_c, s_w, d_w):
    import jax.experimental.pallas.tpu_sc as plsc
    e = s_c.shape[0]
    info = plsc.get_sparse_core_info()
    ns = info.num_subcores
    epw = e // ns

    @functools.partial(
        pl.kernel,
        out_type=[jax.ShapeDtypeStruct((2 * e, 512), jnp.float32),
                  jax.ShapeDtypeStruct((2 * e, 256), jnp.float32)],
        mesh=plsc.VectorSubcoreMesh(core_axis_name="c", subcore_axis_name="s"),
        scratch_types=[
            pltpu.VMEM((SV_GRP * GCH,), jnp.int32),
            pltpu.VMEM((SV_GRP * GCH, 512), jnp.float32),
            pltpu.VMEM((QI_GRP * GCH,), jnp.int32),
            pltpu.VMEM((QI_GRP * GCH, 256), jnp.float32),
            pltpu.SemaphoreType.DMA,
        ],
    )
    def k(svp_h, svw_h, qp_h, sc_h, dc_h, sw_h, dw_h, kjv_h, qi_h,
          idx1, rows1, idx2, rows2, sem):
        c = lax.axis_index("c")
        s = lax.axis_index("s")
        base = s * epw

        @pl.when(c == 0)
        def _():
            _gather_job(sc_h, svp_h, kjv_h, base, base, idx1, rows1, sem,
                        epw, SV_GRP)
            _gather_job(dc_h, qp_h, qi_h, base, base, idx2, rows2, sem,
                        epw, QI_GRP)

        @pl.when(c == 1)
        def _():
            _gather_job(sw_h, svw_h, kjv_h, base, e + base, idx1, rows1, sem,
                        epw, SV_GRP)
            _gather_job(dw_h, qp_h, qi_h, base, e + base, idx2, rows2, sem,
                        epw, QI_GRP)

    return k(svp, svw, qp, s_c, d_c, s_w, d_w)


def _assemble(params):
    pre = {'layers': []}
    for lp in params['layers']:
        comp = {}
        for (src, rel, dst) in ETS:
            scale = lp['p_rel'][rel] / math.sqrt(DH)
            a = lp['a_rel'][rel] * scale[:, None, None]
            m = lp['m_rel'][rel]
            fin = lp['W_k'][src].shape[0]
            wka = jnp.einsum('ihd,hdf->ihf', lp['W_k'][src].reshape(fin, H, DH), a).reshape(fin, HID)
            bka = jnp.einsum('hd,hdf->hf', lp['b_k'][src].reshape(H, DH), a).reshape(HID)
            wvm = jnp.einsum('ihd,hdf->ihf', lp['W_v'][src].reshape(fin, H, DH), m).reshape(fin, HID)
            bvm = jnp.einsum('hd,hdf->hf', lp['b_v'][src].reshape(H, DH), m).reshape(HID)
            comp[rel] = (wka, bka, wvm, bvm)
        wcat_p = jnp.concatenate([comp['cites'][0], comp['cites'][2],
                                  lp['W_q']['paper']], axis=1)
        bcat_p = jnp.concatenate([comp['cites'][1], comp['cites'][3],
                                  lp['b_q']['paper']])
        wcat_a = jnp.concatenate([comp['writes'][0], comp['writes'][2]], axis=1)
        bcat_a = jnp.concatenate([comp['writes'][1], comp['writes'][3]])
        sig_p = jax.nn.sigmoid(lp['skip']['paper'])
        sig_a = jax.nn.sigmoid(lp['skip']['author'])
        ones = jnp.ones((1, 256), jnp.float32)
        pre['layers'].append({
            'wcat_p': wcat_p, 'bcat_p': bcat_p,
            'wcat_a': wcat_a, 'bcat_a': bcat_a,
            'wo_p': lp['W_o']['paper'] * sig_p,
            'bo_p': lp['b_o']['paper'] * sig_p,
            'cp': ones * (1.0 - sig_p),
            'ca': ones * (1.0 - sig_a),
            'da': (lp['b_o']['author'] * sig_a).reshape(1, 256),
        })
    return pre


def kernel(x_paper, x_author, edge_index_cites, edge_index_writes,
           batch_paper, batch_author, params):
    n = x_paper.shape[0]
    e = edge_index_cites.shape[1]
    s_c, d_c = edge_index_cites[0], edge_index_cites[1]
    s_w, d_w = edge_index_writes[0], edge_index_writes[1]
    pre = _assemble(params)

    xp, xa = x_paper, x_author
    xs_p, xs_a = [], []
    nl = len(params['layers'])
    for l in range(nl):
        w = pre['layers'][l]
        svp, qp = _proj_p(xp, w['wcat_p'], w['bcat_p'])
        svw = _matmul(xa, w['wcat_a'], w['bcat_a'])
        kjv, qi = _sc_gather(svp, svw, qp, s_c, d_c, s_w, d_w)
        msgp = _msg(kjv, qi)
        nd_c = jax.ops.segment_sum(msgp[:e], d_c, num_segments=n)
        nd_w = jax.ops.segment_sum(msgp[e:], d_w, num_segments=n)
        xp, xa = _post(nd_c, nd_w, xp,
                       w['wo_p'], w['bo_p'], w['cp'], xa, w['ca'], w['da'])
        if l != nl - 1:
            xp = _graph_ln(xp, params['norm_w']['paper'], params['norm_b']['paper'])
            xa = _graph_ln(xa, params['norm_w']['author'], params['norm_b']['author'])
        xs_p.append(xp)
        xs_a.append(xa)

    out_p = _heads(xs_p[0], xs_p[1], params['out_W']['paper'], params['out_b']['paper'])
    out_a = _heads(xs_a[0], xs_a[1], params['out_W']['author'], params['out_b']['author'])
    return (out_p, out_a)
